# baseline (speedup 1.0000x reference)
import functools

import jax
import jax.numpy as jnp
import numpy as np
from jax import lax
from jax.experimental import pallas as pl
from jax.experimental.pallas import tpu as pltpu, tpu_sc as plsc

EPS = 1e-5
_INTERPRET = False


def _t0_body(h_ref, w1a_ref, w1b_ref, b1_ref, p_ref, q_ref):
    h = h_ref[...]
    p_ref[...] = jnp.dot(h, w1a_ref[...], preferred_element_type=jnp.float32)
    q_ref[...] = (jnp.dot(h, w1b_ref[...], preferred_element_type=jnp.float32)
                  + b1_ref[...])


def _precompute_pq(h, W1a, W1b, b1):
    n, d = h.shape
    return pl.pallas_call(
        _t0_body,
        out_shape=(jax.ShapeDtypeStruct((n, d), jnp.float32),
                   jax.ShapeDtypeStruct((n, d), jnp.float32)),
        interpret=_INTERPRET,
    )(h, W1a, W1b, b1.reshape(1, d))


def _coeffs_from_acc(acc, gb, count):
    mean = acc[0:1, :] * (1.0 / count)
    var = acc[1:2, :] * (1.0 / count) - mean * mean
    s = gb[0:1, :] * lax.rsqrt(var + EPS)
    return jnp.concatenate([s, gb[1:2, :] - mean * s], axis=0)


def _ta_body(t_ref, ea_ref, w1c_ref, gb_ref, y1_ref, ss_ref, acc_ref):
    i = pl.program_id(0)
    y = t_ref[...] + jnp.dot(
        ea_ref[...], w1c_ref[...], preferred_element_type=jnp.float32)
    y1_ref[...] = y.astype(jnp.bfloat16)
    ones = jnp.full((1, y.shape[0]), 1.0, jnp.float32)
    s = jnp.concatenate(
        [jnp.dot(ones, y, preferred_element_type=jnp.float32),
         jnp.dot(ones, y * y, preferred_element_type=jnp.float32)], axis=0)

    @pl.when(i == 0)
    def _():
        acc_ref[...] = s

    @pl.when(i > 0)
    def _():
        acc_ref[...] = acc_ref[...] + s

    @pl.when(i == pl.num_programs(0) - 1)
    def _():
        ss_ref[...] = _coeffs_from_acc(acc_ref[...], gb_ref[...],
                                       t_ref.shape[0] * pl.num_programs(0))


def _pass_a(t, ea, W1c, g1, be1, eb):
    e, d = t.shape
    de = ea.shape[1]
    grid = (e // eb,)
    gb = jnp.stack([g1, be1])
    return pl.pallas_call(
        _ta_body,
        grid=grid,
        in_specs=[
            pl.BlockSpec((eb, d), lambda i: (i, 0)),
            pl.BlockSpec((eb, de), lambda i: (i, 0)),
            pl.BlockSpec((de, d), lambda i: (0, 0)),
            pl.BlockSpec((2, d), lambda i: (0, 0)),
        ],
        out_specs=[
            pl.BlockSpec((eb, d), lambda i: (i, 0)),
            pl.BlockSpec((2, d), lambda i: (0, 0)),
        ],
        out_shape=(jax.ShapeDtypeStruct((e, d), jnp.bfloat16),
                   jax.ShapeDtypeStruct((2, d), jnp.float32)),
        scratch_shapes=[pltpu.VMEM((2, d), jnp.float32)],
        interpret=_INTERPRET,
    )(t, ea, W1c, gb)


def _tb_body(y1_ref, ss1_ref, w2_ref, b2_ref, gb_ref,
             y2_ref, ss_ref, acc_ref):
    i = pl.program_id(0)
    y1 = y1_ref[...].astype(jnp.float32)
    a = jnp.maximum(y1 * ss1_ref[0:1, :] + ss1_ref[1:2, :], 0.0)
    y = jnp.dot(a, w2_ref[...], preferred_element_type=jnp.float32) + b2_ref[...]
    y2_ref[...] = y
    ones = jnp.full((1, y.shape[0]), 1.0, jnp.float32)
    s = jnp.concatenate(
        [jnp.dot(ones, y, preferred_element_type=jnp.float32),
         jnp.dot(ones, y * y, preferred_element_type=jnp.float32)], axis=0)

    @pl.when(i == 0)
    def _():
        acc_ref[...] = s

    @pl.when(i > 0)
    def _():
        acc_ref[...] = acc_ref[...] + s

    @pl.when(i == pl.num_programs(0) - 1)
    def _():
        ss_ref[...] = _coeffs_from_acc(acc_ref[...], gb_ref[...],
                                       y1_ref.shape[0] * pl.num_programs(0))


def _pass_b(y1, ss1, W2, b2, g2, be2, eb):
    e, d = y1.shape
    grid = (e // eb,)
    gb = jnp.stack([g2, be2])
    return pl.pallas_call(
        _tb_body,
        grid=grid,
        in_specs=[
            pl.BlockSpec((eb, d), lambda i: (i, 0)),
            pl.BlockSpec((2, d), lambda i: (0, 0)),
            pl.BlockSpec((d, d), lambda i: (0, 0)),
            pl.BlockSpec((1, d), lambda i: (0, 0)),
            pl.BlockSpec((2, d), lambda i: (0, 0)),
        ],
        out_specs=[
            pl.BlockSpec((eb, d), lambda i: (i, 0)),
            pl.BlockSpec((2, d), lambda i: (0, 0)),
        ],
        out_shape=(jax.ShapeDtypeStruct((e, d), jnp.float32),
                   jax.ShapeDtypeStruct((2, d), jnp.float32)),
        scratch_shapes=[pltpu.VMEM((2, d), jnp.float32)],
        interpret=_INTERPRET,
    )(y1, ss1, W2, b2.reshape(1, d), gb)


def _tn_body(h_ref, a0_ref, a1_ref, u1a_ref, u1b_ref, ub1_ref, g1_ref, be1_ref,
             u2_ref, ub2_ref, g2_ref, be2_ref, out_ref):
    n = h_ref.shape[0]
    inv_n = 1.0 / n
    aggr = a0_ref[0:n, :] + a1_ref[0:n, :]
    y = (jnp.dot(h_ref[...], u1a_ref[...], preferred_element_type=jnp.float32)
         + jnp.dot(aggr, u1b_ref[...], preferred_element_type=jnp.float32)
         + ub1_ref[...])
    ones = jnp.full((1, n), 1.0, jnp.float32)
    m = jnp.dot(ones, y, preferred_element_type=jnp.float32) * inv_n
    v = (jnp.dot(ones, y * y, preferred_element_type=jnp.float32) * inv_n
         - m * m)
    s = g1_ref[...] * lax.rsqrt(v + EPS)
    a = jnp.maximum(y * s + (be1_ref[...] - m * s), 0.0)
    y = (jnp.dot(a, u2_ref[...], preferred_element_type=jnp.float32)
         + ub2_ref[...])
    m = jnp.dot(ones, y, preferred_element_type=jnp.float32) * inv_n
    v = (jnp.dot(ones, y * y, preferred_element_type=jnp.float32) * inv_n
         - m * m)
    s = g2_ref[...] * lax.rsqrt(v + EPS)
    out_ref[...] = jnp.maximum(y * s + (be2_ref[...] - m * s), 0.0)


def _node_mlp(h, a0, a1, U1a, U1b, ub1, ug1, ube1, U2, ub2, ug2, ube2):
    n, d = h.shape
    r = lambda x: x.reshape(1, d)
    return pl.pallas_call(
        _tn_body,
        out_shape=jax.ShapeDtypeStruct((n, d), jnp.float32),
        interpret=_INTERPRET,
    )(h, a0, a1, U1a, U1b, r(ub1), r(ug1), r(ube1), U2, r(ub2), r(ug2), r(ube2))


_NC, _NS, _L = 2, 16, 16
_NW = _NC * _NS
_C = 80
_NBUF = 5


def _copy_idx_chunk(idx_all, off, dst_row):
    for k in range(_C // _L):
        sl = pl.ds(k * _L, _L)
        dst_row[sl] = idx_all[pl.ds(off + k * _L, _L)]


_NBG = 4

def _scg_body(p_hbm, q_hbm, src_hbm, dst_hbm, t_hbm,
              ia_s, ia_d, ib_s, ib_d, bufp, bufq, sem_g, sem_w):
    e = t_hbm.shape[0]
    d = p_hbm.shape[1]
    per_w = e // _NW
    nchunks = per_w // _C
    wid = lax.axis_index("s") * _NC + lax.axis_index("c")
    w0 = wid * per_w

    pltpu.sync_copy(src_hbm.at[pl.ds(w0, per_w)], ia_s)
    pltpu.sync_copy(dst_hbm.at[pl.ds(w0, per_w)], ia_d)

    def fire_gather(j, b):
        _copy_idx_chunk(ia_s, j * _C, ib_s.at[b])
        _copy_idx_chunk(ia_d, j * _C, ib_d.at[b])
        pltpu.async_copy(p_hbm.at[ib_s.at[b]], bufp.at[b], sem_g.at[b])
        pltpu.async_copy(q_hbm.at[ib_d.at[b]], bufq.at[b], sem_g.at[b])

    def drain_gather(b):
        pltpu.make_async_copy(p_hbm.at[ib_s.at[b]], bufp.at[b],
                              sem_g.at[b]).wait()
        pltpu.make_async_copy(q_hbm.at[ib_d.at[b]], bufq.at[b],
                              sem_g.at[b]).wait()

    def compute(b):
        def row(r, c2):
            for u in range(2):
                for k in range(d // _L):
                    sl = pl.ds(k * _L, _L)
                    bufp[b, 2 * r + u, sl] = (bufp[b, 2 * r + u, sl]
                                              + bufq[b, 2 * r + u, sl])
            return c2

        lax.fori_loop(0, _C // 2, row, 0)

    def fire_write(j, b):
        pltpu.async_copy(bufp.at[b], t_hbm.at[pl.ds(w0 + j * _C, _C)],
                         sem_w.at[b])

    def drain_write(j, b):
        pltpu.make_async_copy(bufp.at[b], t_hbm.at[pl.ds(w0 + j * _C, _C)],
                              sem_w.at[b]).wait()

    for b in range(_NBG - 1):
        fire_gather(b, b)

    def outer(o, carry):
        for b in range(_NBG):
            j = o * _NBG + b
            drain_gather(b)
            compute(b)

            @pl.when(j >= 1)
            def _():
                drain_write(j - 1, (b - 1) % _NBG)

            @pl.when(j + _NBG - 1 < nchunks)
            def _():
                fire_gather(j + _NBG - 1, (b + _NBG - 1) % _NBG)

            fire_write(j, b)
        return carry

    ntail = nchunks % _NBG
    lax.fori_loop(0, nchunks // _NBG, outer, 0)
    for x in range(ntail):
        j = nchunks - ntail + x
        b = j % _NBG
        drain_gather(b)
        compute(b)
        drain_write(j - 1, (b - 1) % _NBG)
        fire_write(j, b)
    drain_write(nchunks - 1, (nchunks - 1) % _NBG)


def _sc_gather_add(p, q, src, dst):
    n, d = p.shape
    e = src.shape[0]
    per_w = e // _NW
    mesh = plsc.VectorSubcoreMesh(core_axis_name="c", subcore_axis_name="s")
    return pl.kernel(
        _scg_body,
        out_type=jax.ShapeDtypeStruct((e, d), jnp.float32),
        mesh=mesh,
        scratch_types=[
            pltpu.VMEM((per_w,), jnp.int32),
            pltpu.VMEM((per_w,), jnp.int32),
            pltpu.VMEM((_NBG, _C), jnp.int32),
            pltpu.VMEM((_NBG, _C), jnp.int32),
            pltpu.VMEM((_NBG, _C, d), jnp.float32),
            pltpu.VMEM((_NBG, _C, d), jnp.float32),
            pltpu.SemaphoreType.DMA((_NBG,)),
            pltpu.SemaphoreType.DMA((_NBG,)),
        ],
    )(p, q, src, dst)


_NBS = 4


def _scs_body(y2_hbm, src_hbm, st_hbm, part_hbm,
              ib_s, buf, stv, aggr_sh, sem_l, sem_sc):
    e = y2_hbm.shape[0]
    np_ = part_hbm.shape[1]
    d = y2_hbm.shape[1]
    per_w = e // _NW
    nchunks = per_w // _C
    rpt = np_ // _NS
    sid = lax.axis_index("s")
    cid = lax.axis_index("c")
    wid = sid * _NC + cid
    w0 = wid * per_w

    zvec = jnp.zeros((_L,), jnp.float32)

    def zrow(r, c2):
        for u in range(2):
            for k in range(d // _L):
                buf[0, 2 * r + u, pl.ds(k * _L, _L)] = zvec
        return c2

    lax.fori_loop(0, _C // 2, zrow, 0)
    for i in range(rpt // _C):
        pltpu.sync_copy(buf.at[0],
                        aggr_sh.at[pl.ds(sid * rpt + i * _C, _C)])
    rem = rpt % _C
    if rem:
        pltpu.sync_copy(buf.at[0, :rem],
                        aggr_sh.at[pl.ds(sid * rpt + rpt - rem, rem)])
    pltpu.sync_copy(st_hbm, stv)
    plsc.subcore_barrier()
    svec = [stv[0, pl.ds(k * _L, _L)] for k in range(d // _L)]
    tvec = [stv[1, pl.ds(k * _L, _L)] for k in range(d // _L)]

    def fire_load(j, b):
        pltpu.async_copy(y2_hbm.at[pl.ds(w0 + j * _C, _C)], buf.at[b],
                         sem_l.at[b])
        pltpu.async_copy(src_hbm.at[pl.ds(w0 + j * _C, _C)], ib_s.at[b],
                         sem_l.at[b])

    def drain_load(j, b):
        pltpu.make_async_copy(y2_hbm.at[pl.ds(w0 + j * _C, _C)],
                              buf.at[b], sem_l.at[b]).wait()
        pltpu.make_async_copy(src_hbm.at[pl.ds(w0 + j * _C, _C)],
                              ib_s.at[b], sem_l.at[b]).wait()

    def compute(b):
        def row(r, c2):
            for u in range(2):
                for k in range(d // _L):
                    sl = pl.ds(k * _L, _L)
                    buf[b, 2 * r + u, sl] = jnp.maximum(
                        buf[b, 2 * r + u, sl] * svec[k] + tvec[k], 0.0)
            return c2

        lax.fori_loop(0, _C // 2, row, 0)

    def drain_scatter(b):
        pltpu.make_async_copy(buf.at[b], aggr_sh.at[ib_s.at[b]],
                              sem_sc).wait()

    for b in range(_NBS - 1):
        fire_load(b, b)

    def outer(o, carry):
        for b in range(_NBS):
            j = o * _NBS + b
            drain_load(j, b)
            compute(b)

            @pl.when(j >= 1)
            def _():
                drain_scatter((b - 1) % _NBS)

            @pl.when(j + _NBS - 1 < nchunks)
            def _():
                fire_load(j + _NBS - 1, (b + _NBS - 1) % _NBS)

            pltpu.async_copy(buf.at[b], aggr_sh.at[ib_s.at[b]], sem_sc,
                             add=True)
        return carry

    ntail = nchunks % _NBS
    lax.fori_loop(0, nchunks // _NBS, outer, 0)
    for t in range(ntail):
        j = nchunks - ntail + t
        b = j % _NBS
        drain_load(j, b)
        compute(b)
        drain_scatter((b - 1) % _NBS)
        pltpu.async_copy(buf.at[b], aggr_sh.at[ib_s.at[b]], sem_sc,
                         add=True)
    drain_scatter((nchunks - 1) % _NBS)
    plsc.subcore_barrier()
    pltpu.sync_copy(aggr_sh.at[pl.ds(sid * rpt, rpt)],
                    part_hbm.at[cid, pl.ds(sid * rpt, rpt)])


def _sc_scatter(y2, src, ss2, n):
    e, d = y2.shape
    np_ = ((n + 8 * _NS - 1) // (8 * _NS)) * (8 * _NS)
    mesh = plsc.VectorSubcoreMesh(core_axis_name="c", subcore_axis_name="s")
    return pl.kernel(
        _scs_body,
        out_type=jax.ShapeDtypeStruct((_NC, np_, d), jnp.float32),
        mesh=mesh,
        scratch_types=[
            pltpu.VMEM((_NBS, _C), jnp.int32),
            pltpu.VMEM((_NBS, _C, d), jnp.float32),
            pltpu.VMEM((2, d), jnp.float32),
            pltpu.VMEM_SHARED((np_, d), jnp.float32),
            pltpu.SemaphoreType.DMA((_NBS,)),
            pltpu.SemaphoreType.DMA,
        ],
    )(y2, src, ss2)


def kernel(h, edge_index, edge_attr, W1, b1, g1, be1, W2, b2, g2, be2,
           U1, ub1, ug1, ube1, U2, ub2, ug2, ube2):
    n, d = h.shape
    e = edge_index.shape[1]
    src = edge_index[0].astype(jnp.int32)
    dst = edge_index[1].astype(jnp.int32)
    W1a, W1b, W1c = W1[:d], W1[d:2 * d], W1[2 * d:]
    EB = 8000

    p, q = _precompute_pq(h, W1a, W1b, b1)
    t = _sc_gather_add(p, q, src, dst)
    y1, ss1 = _pass_a(t, edge_attr, W1c, g1, be1, EB)
    y2, ss2 = _pass_b(y1, ss1, W2, b2, g2, be2, EB)
    part = _sc_scatter(y2, src, ss2, n)
    return _node_mlp(h, part[0], part[1], U1[:d], U1[d:], ub1, ug1, ube1,
                     U2, ub2, ug2, ube2)


# speedup vs baseline: 1.0343x; 1.0343x over previous
import functools

import jax
import jax.numpy as jnp
import numpy as np
from jax import lax
from jax.experimental import pallas as pl
from jax.experimental.pallas import tpu as pltpu, tpu_sc as plsc

EPS = 1e-5
_INTERPRET = False


def _t0_body(h_ref, w1a_ref, w1b_ref, b1_ref, p_ref, q_ref):
    h = h_ref[...]
    p_ref[...] = jnp.dot(h, w1a_ref[...], preferred_element_type=jnp.float32)
    q_ref[...] = (jnp.dot(h, w1b_ref[...], preferred_element_type=jnp.float32)
                  + b1_ref[...])


def _precompute_pq(h, W1a, W1b, b1):
    n, d = h.shape
    return pl.pallas_call(
        _t0_body,
        out_shape=(jax.ShapeDtypeStruct((n, d), jnp.float32),
                   jax.ShapeDtypeStruct((n, d), jnp.float32)),
        interpret=_INTERPRET,
    )(h, W1a, W1b, b1.reshape(1, d))


def _coeffs_from_acc(acc, gb, count):
    mean = acc[0:1, :] * (1.0 / count)
    var = acc[1:2, :] * (1.0 / count) - mean * mean
    s = gb[0:1, :] * lax.rsqrt(var + EPS)
    return jnp.concatenate([s, gb[1:2, :] - mean * s], axis=0)


def _ta_body(t_ref, ea_ref, w1c_ref, gb_ref, y1_ref, ss_ref, acc_ref):
    i = pl.program_id(0)
    y = t_ref[...] + jnp.dot(
        ea_ref[...], w1c_ref[...], preferred_element_type=jnp.float32)
    y1_ref[...] = y.astype(jnp.bfloat16)
    s = jnp.concatenate([jnp.sum(y, 0, keepdims=True),
                         jnp.sum(y * y, 0, keepdims=True)], axis=0)

    @pl.when(i == 0)
    def _():
        acc_ref[...] = s

    @pl.when(i > 0)
    def _():
        acc_ref[...] = acc_ref[...] + s

    @pl.when(i == pl.num_programs(0) - 1)
    def _():
        ss_ref[...] = _coeffs_from_acc(acc_ref[...], gb_ref[...],
                                       t_ref.shape[0] * pl.num_programs(0))


def _pass_a(t, ea, W1c, g1, be1, eb):
    e, d = t.shape
    de = ea.shape[1]
    grid = (e // eb,)
    gb = jnp.stack([g1, be1])
    return pl.pallas_call(
        _ta_body,
        grid=grid,
        in_specs=[
            pl.BlockSpec((eb, d), lambda i: (i, 0)),
            pl.BlockSpec((eb, de), lambda i: (i, 0)),
            pl.BlockSpec((de, d), lambda i: (0, 0)),
            pl.BlockSpec((2, d), lambda i: (0, 0)),
        ],
        out_specs=[
            pl.BlockSpec((eb, d), lambda i: (i, 0)),
            pl.BlockSpec((2, d), lambda i: (0, 0)),
        ],
        out_shape=(jax.ShapeDtypeStruct((e, d), jnp.bfloat16),
                   jax.ShapeDtypeStruct((2, d), jnp.float32)),
        scratch_shapes=[pltpu.VMEM((2, d), jnp.float32)],
        interpret=_INTERPRET,
    )(t, ea, W1c, gb)


def _tb_body(y1_ref, ss1_ref, w2_ref, b2_ref, gb_ref,
             y2_ref, ss_ref, acc_ref):
    i = pl.program_id(0)
    y1 = y1_ref[...].astype(jnp.float32)
    a = jnp.maximum(y1 * ss1_ref[0:1, :] + ss1_ref[1:2, :], 0.0)
    y = jnp.dot(a, w2_ref[...], preferred_element_type=jnp.float32) + b2_ref[...]
    y2_ref[...] = y
    s = jnp.concatenate([jnp.sum(y, 0, keepdims=True),
                         jnp.sum(y * y, 0, keepdims=True)], axis=0)

    @pl.when(i == 0)
    def _():
        acc_ref[...] = s

    @pl.when(i > 0)
    def _():
        acc_ref[...] = acc_ref[...] + s

    @pl.when(i == pl.num_programs(0) - 1)
    def _():
        ss_ref[...] = _coeffs_from_acc(acc_ref[...], gb_ref[...],
                                       y1_ref.shape[0] * pl.num_programs(0))


def _pass_b(y1, ss1, W2, b2, g2, be2, eb):
    e, d = y1.shape
    grid = (e // eb,)
    gb = jnp.stack([g2, be2])
    return pl.pallas_call(
        _tb_body,
        grid=grid,
        in_specs=[
            pl.BlockSpec((eb, d), lambda i: (i, 0)),
            pl.BlockSpec((2, d), lambda i: (0, 0)),
            pl.BlockSpec((d, d), lambda i: (0, 0)),
            pl.BlockSpec((1, d), lambda i: (0, 0)),
            pl.BlockSpec((2, d), lambda i: (0, 0)),
        ],
        out_specs=[
            pl.BlockSpec((eb, d), lambda i: (i, 0)),
            pl.BlockSpec((2, d), lambda i: (0, 0)),
        ],
        out_shape=(jax.ShapeDtypeStruct((e, d), jnp.float32),
                   jax.ShapeDtypeStruct((2, d), jnp.float32)),
        scratch_shapes=[pltpu.VMEM((2, d), jnp.float32)],
        interpret=_INTERPRET,
    )(y1, ss1, W2, b2.reshape(1, d), gb)


def _tn_body(h_ref, a0_ref, a1_ref, u1a_ref, u1b_ref, ub1_ref, g1_ref, be1_ref,
             u2_ref, ub2_ref, g2_ref, be2_ref, out_ref):
    n = h_ref.shape[0]
    inv_n = 1.0 / n
    aggr = a0_ref[0:n, :] + a1_ref[0:n, :]
    y = (jnp.dot(h_ref[...], u1a_ref[...], preferred_element_type=jnp.float32)
         + jnp.dot(aggr, u1b_ref[...], preferred_element_type=jnp.float32)
         + ub1_ref[...])
    m = jnp.sum(y, 0, keepdims=True) * inv_n
    v = jnp.sum(y * y, 0, keepdims=True) * inv_n - m * m
    s = g1_ref[...] * lax.rsqrt(v + EPS)
    a = jnp.maximum(y * s + (be1_ref[...] - m * s), 0.0)
    y = (jnp.dot(a, u2_ref[...], preferred_element_type=jnp.float32)
         + ub2_ref[...])
    m = jnp.sum(y, 0, keepdims=True) * inv_n
    v = jnp.sum(y * y, 0, keepdims=True) * inv_n - m * m
    s = g2_ref[...] * lax.rsqrt(v + EPS)
    out_ref[...] = jnp.maximum(y * s + (be2_ref[...] - m * s), 0.0)


def _node_mlp(h, a0, a1, U1a, U1b, ub1, ug1, ube1, U2, ub2, ug2, ube2):
    n, d = h.shape
    r = lambda x: x.reshape(1, d)
    return pl.pallas_call(
        _tn_body,
        out_shape=jax.ShapeDtypeStruct((n, d), jnp.float32),
        interpret=_INTERPRET,
    )(h, a0, a1, U1a, U1b, r(ub1), r(ug1), r(ube1), U2, r(ub2), r(ug2), r(ube2))


_NC, _NS, _L = 2, 16, 16
_NW = _NC * _NS
_C = 80
_NBUF = 5


def _copy_idx_chunk(idx_all, off, dst_row):
    for k in range(_C // _L):
        sl = pl.ds(k * _L, _L)
        dst_row[sl] = idx_all[pl.ds(off + k * _L, _L)]


_NBG = 4

def _scg_body(p_hbm, q_hbm, src_hbm, dst_hbm, t_hbm,
              ia_s, ia_d, ib_s, ib_d, bufp, bufq, sem_g, sem_w):
    e = t_hbm.shape[0]
    d = p_hbm.shape[1]
    per_w = e // _NW
    nchunks = per_w // _C
    wid = lax.axis_index("s") * _NC + lax.axis_index("c")
    w0 = wid * per_w

    pltpu.sync_copy(src_hbm.at[pl.ds(w0, per_w)], ia_s)
    pltpu.sync_copy(dst_hbm.at[pl.ds(w0, per_w)], ia_d)

    def fire_gather(j, b):
        _copy_idx_chunk(ia_s, j * _C, ib_s.at[b])
        _copy_idx_chunk(ia_d, j * _C, ib_d.at[b])
        pltpu.async_copy(p_hbm.at[ib_s.at[b]], bufp.at[b], sem_g.at[b])
        pltpu.async_copy(q_hbm.at[ib_d.at[b]], bufq.at[b], sem_g.at[b])

    def drain_gather(b):
        pltpu.make_async_copy(p_hbm.at[ib_s.at[b]], bufp.at[b],
                              sem_g.at[b]).wait()
        pltpu.make_async_copy(q_hbm.at[ib_d.at[b]], bufq.at[b],
                              sem_g.at[b]).wait()

    def compute(b):
        def row(r, c2):
            for u in range(2):
                for k in range(d // _L):
                    sl = pl.ds(k * _L, _L)
                    bufp[b, 2 * r + u, sl] = (bufp[b, 2 * r + u, sl]
                                              + bufq[b, 2 * r + u, sl])
            return c2

        lax.fori_loop(0, _C // 2, row, 0)

    def fire_write(j, b):
        pltpu.async_copy(bufp.at[b], t_hbm.at[pl.ds(w0 + j * _C, _C)],
                         sem_w.at[b])

    def drain_write(j, b):
        pltpu.make_async_copy(bufp.at[b], t_hbm.at[pl.ds(w0 + j * _C, _C)],
                              sem_w.at[b]).wait()

    for b in range(_NBG - 1):
        fire_gather(b, b)

    def outer(o, carry):
        for b in range(_NBG):
            j = o * _NBG + b
            drain_gather(b)
            compute(b)

            @pl.when(j >= 1)
            def _():
                drain_write(j - 1, (b - 1) % _NBG)

            @pl.when(j + _NBG - 1 < nchunks)
            def _():
                fire_gather(j + _NBG - 1, (b + _NBG - 1) % _NBG)

            fire_write(j, b)
        return carry

    ntail = nchunks % _NBG
    lax.fori_loop(0, nchunks // _NBG, outer, 0)
    for x in range(ntail):
        j = nchunks - ntail + x
        b = j % _NBG
        drain_gather(b)
        compute(b)
        drain_write(j - 1, (b - 1) % _NBG)
        fire_write(j, b)
    drain_write(nchunks - 1, (nchunks - 1) % _NBG)


def _sc_gather_add(p, q, src, dst):
    n, d = p.shape
    e = src.shape[0]
    per_w = e // _NW
    mesh = plsc.VectorSubcoreMesh(core_axis_name="c", subcore_axis_name="s")
    return pl.kernel(
        _scg_body,
        out_type=jax.ShapeDtypeStruct((e, d), jnp.float32),
        mesh=mesh,
        scratch_types=[
            pltpu.VMEM((per_w,), jnp.int32),
            pltpu.VMEM((per_w,), jnp.int32),
            pltpu.VMEM((_NBG, _C), jnp.int32),
            pltpu.VMEM((_NBG, _C), jnp.int32),
            pltpu.VMEM((_NBG, _C, d), jnp.float32),
            pltpu.VMEM((_NBG, _C, d), jnp.float32),
            pltpu.SemaphoreType.DMA((_NBG,)),
            pltpu.SemaphoreType.DMA((_NBG,)),
        ],
    )(p, q, src, dst)


_NBS = 4


def _scs_body(y2_hbm, src_hbm, st_hbm, part_hbm,
              ib_s, buf, stv, aggr_sh, sem_l, sem_sc):
    e = y2_hbm.shape[0]
    np_ = part_hbm.shape[1]
    d = y2_hbm.shape[1]
    per_w = e // _NW
    nchunks = per_w // _C
    rpt = np_ // _NS
    sid = lax.axis_index("s")
    cid = lax.axis_index("c")
    wid = sid * _NC + cid
    w0 = wid * per_w

    zvec = jnp.zeros((_L,), jnp.float32)

    def zrow(r, c2):
        for u in range(2):
            for k in range(d // _L):
                buf[0, 2 * r + u, pl.ds(k * _L, _L)] = zvec
        return c2

    lax.fori_loop(0, _C // 2, zrow, 0)
    for i in range(rpt // _C):
        pltpu.sync_copy(buf.at[0],
                        aggr_sh.at[pl.ds(sid * rpt + i * _C, _C)])
    rem = rpt % _C
    if rem:
        pltpu.sync_copy(buf.at[0, :rem],
                        aggr_sh.at[pl.ds(sid * rpt + rpt - rem, rem)])
    pltpu.sync_copy(st_hbm, stv)
    plsc.subcore_barrier()
    svec = [stv[0, pl.ds(k * _L, _L)] for k in range(d // _L)]
    tvec = [stv[1, pl.ds(k * _L, _L)] for k in range(d // _L)]

    def fire_load(j, b):
        pltpu.async_copy(y2_hbm.at[pl.ds(w0 + j * _C, _C)], buf.at[b],
                         sem_l.at[b])
        pltpu.async_copy(src_hbm.at[pl.ds(w0 + j * _C, _C)], ib_s.at[b],
                         sem_l.at[b])

    def drain_load(j, b):
        pltpu.make_async_copy(y2_hbm.at[pl.ds(w0 + j * _C, _C)],
                              buf.at[b], sem_l.at[b]).wait()
        pltpu.make_async_copy(src_hbm.at[pl.ds(w0 + j * _C, _C)],
                              ib_s.at[b], sem_l.at[b]).wait()

    def compute(b):
        def row(r, c2):
            for u in range(2):
                for k in range(d // _L):
                    sl = pl.ds(k * _L, _L)
                    buf[b, 2 * r + u, sl] = jnp.maximum(
                        buf[b, 2 * r + u, sl] * svec[k] + tvec[k], 0.0)
            return c2

        lax.fori_loop(0, _C // 2, row, 0)

    def drain_scatter(b):
        pltpu.make_async_copy(buf.at[b], aggr_sh.at[ib_s.at[b]],
                              sem_sc).wait()

    for b in range(_NBS - 1):
        fire_load(b, b)

    def outer(o, carry):
        for b in range(_NBS):
            j = o * _NBS + b
            drain_load(j, b)
            compute(b)

            @pl.when(j >= 1)
            def _():
                drain_scatter((b - 1) % _NBS)

            @pl.when(j + _NBS - 1 < nchunks)
            def _():
                fire_load(j + _NBS - 1, (b + _NBS - 1) % _NBS)

            pltpu.async_copy(buf.at[b], aggr_sh.at[ib_s.at[b]], sem_sc,
                             add=True)
        return carry

    ntail = nchunks % _NBS
    lax.fori_loop(0, nchunks // _NBS, outer, 0)
    for t in range(ntail):
        j = nchunks - ntail + t
        b = j % _NBS
        drain_load(j, b)
        compute(b)
        drain_scatter((b - 1) % _NBS)
        pltpu.async_copy(buf.at[b], aggr_sh.at[ib_s.at[b]], sem_sc,
                         add=True)
    drain_scatter((nchunks - 1) % _NBS)
    plsc.subcore_barrier()
    pltpu.sync_copy(aggr_sh.at[pl.ds(sid * rpt, rpt)],
                    part_hbm.at[cid, pl.ds(sid * rpt, rpt)])


def _sc_scatter(y2, src, ss2, n):
    e, d = y2.shape
    np_ = ((n + 8 * _NS - 1) // (8 * _NS)) * (8 * _NS)
    mesh = plsc.VectorSubcoreMesh(core_axis_name="c", subcore_axis_name="s")
    return pl.kernel(
        _scs_body,
        out_type=jax.ShapeDtypeStruct((_NC, np_, d), jnp.float32),
        mesh=mesh,
        scratch_types=[
            pltpu.VMEM((_NBS, _C), jnp.int32),
            pltpu.VMEM((_NBS, _C, d), jnp.float32),
            pltpu.VMEM((2, d), jnp.float32),
            pltpu.VMEM_SHARED((np_, d), jnp.float32),
            pltpu.SemaphoreType.DMA((_NBS,)),
            pltpu.SemaphoreType.DMA,
        ],
    )(y2, src, ss2)


def kernel(h, edge_index, edge_attr, W1, b1, g1, be1, W2, b2, g2, be2,
           U1, ub1, ug1, ube1, U2, ub2, ug2, ube2):
    n, d = h.shape
    e = edge_index.shape[1]
    src = edge_index[0].astype(jnp.int32)
    dst = edge_index[1].astype(jnp.int32)
    W1a, W1b, W1c = W1[:d], W1[d:2 * d], W1[2 * d:]
    EB = 16000

    p, q = _precompute_pq(h, W1a, W1b, b1)
    t = _sc_gather_add(p, q, src, dst)
    y1, ss1 = _pass_a(t, edge_attr, W1c, g1, be1, EB)
    y2, ss2 = _pass_b(y1, ss1, W2, b2, g2, be2, EB)
    part = _sc_scatter(y2, src, ss2, n)
    return _node_mlp(h, part[0], part[1], U1[:d], U1[d:], ub1, ug1, ube1,
                     U2, ub2, ug2, ube2)


# speedup vs baseline: 1.0363x; 1.0019x over previous
import functools

import jax
import jax.numpy as jnp
import numpy as np
from jax import lax
from jax.experimental import pallas as pl
from jax.experimental.pallas import tpu as pltpu, tpu_sc as plsc

EPS = 1e-5
_INTERPRET = False


def _t0_body(h_ref, w1a_ref, w1b_ref, b1_ref, p_ref, q_ref):
    h = h_ref[...]
    p_ref[...] = jnp.dot(h, w1a_ref[...], preferred_element_type=jnp.float32)
    q_ref[...] = (jnp.dot(h, w1b_ref[...], preferred_element_type=jnp.float32)
                  + b1_ref[...])


def _precompute_pq(h, W1a, W1b, b1):
    n, d = h.shape
    return pl.pallas_call(
        _t0_body,
        out_shape=(jax.ShapeDtypeStruct((n, d), jnp.float32),
                   jax.ShapeDtypeStruct((n, d), jnp.float32)),
        interpret=_INTERPRET,
    )(h, W1a, W1b, b1.reshape(1, d))


def _coeffs_from_acc(acc, gb, count):
    mean = acc[0:1, :] * (1.0 / count)
    var = acc[1:2, :] * (1.0 / count) - mean * mean
    s = gb[0:1, :] * lax.rsqrt(var + EPS)
    return jnp.concatenate([s, gb[1:2, :] - mean * s], axis=0)


def _ta_body(t_ref, ea_ref, w1c_ref, gb_ref, y1_ref, ss_ref, acc_ref):
    i = pl.program_id(0)
    y = t_ref[...] + jnp.dot(
        ea_ref[...], w1c_ref[...], preferred_element_type=jnp.float32)
    y1_ref[...] = y.astype(jnp.bfloat16)
    s = jnp.concatenate([jnp.sum(y, 0, keepdims=True),
                         jnp.sum(y * y, 0, keepdims=True)], axis=0)

    @pl.when(i == 0)
    def _():
        acc_ref[...] = s

    @pl.when(i > 0)
    def _():
        acc_ref[...] = acc_ref[...] + s

    @pl.when(i == pl.num_programs(0) - 1)
    def _():
        ss_ref[...] = _coeffs_from_acc(acc_ref[...], gb_ref[...],
                                       t_ref.shape[0] * pl.num_programs(0))


def _pass_a(t, ea, W1c, g1, be1, eb):
    e, d = t.shape
    de = ea.shape[1]
    grid = (e // eb,)
    gb = jnp.stack([g1, be1])
    return pl.pallas_call(
        _ta_body,
        grid=grid,
        in_specs=[
            pl.BlockSpec((eb, d), lambda i: (i, 0)),
            pl.BlockSpec((eb, de), lambda i: (i, 0)),
            pl.BlockSpec((de, d), lambda i: (0, 0)),
            pl.BlockSpec((2, d), lambda i: (0, 0)),
        ],
        out_specs=[
            pl.BlockSpec((eb, d), lambda i: (i, 0)),
            pl.BlockSpec((2, d), lambda i: (0, 0)),
        ],
        out_shape=(jax.ShapeDtypeStruct((e, d), jnp.bfloat16),
                   jax.ShapeDtypeStruct((2, d), jnp.float32)),
        scratch_shapes=[pltpu.VMEM((2, d), jnp.float32)],
        interpret=_INTERPRET,
    )(t, ea, W1c, gb)


def _tb_body(y1_ref, ss1_ref, w2_ref, b2_ref, gb_ref,
             y2_ref, ss_ref, acc_ref):
    i = pl.program_id(0)
    y1 = y1_ref[...].astype(jnp.float32)
    a = jnp.maximum(y1 * ss1_ref[0:1, :] + ss1_ref[1:2, :], 0.0)
    y = jnp.dot(a, w2_ref[...], preferred_element_type=jnp.float32) + b2_ref[...]
    y2_ref[...] = y
    s = jnp.concatenate([jnp.sum(y, 0, keepdims=True),
                         jnp.sum(y * y, 0, keepdims=True)], axis=0)

    @pl.when(i == 0)
    def _():
        acc_ref[...] = s

    @pl.when(i > 0)
    def _():
        acc_ref[...] = acc_ref[...] + s

    @pl.when(i == pl.num_programs(0) - 1)
    def _():
        ss_ref[...] = _coeffs_from_acc(acc_ref[...], gb_ref[...],
                                       y1_ref.shape[0] * pl.num_programs(0))


def _pass_b(y1, ss1, W2, b2, g2, be2, eb):
    e, d = y1.shape
    grid = (e // eb,)
    gb = jnp.stack([g2, be2])
    return pl.pallas_call(
        _tb_body,
        grid=grid,
        in_specs=[
            pl.BlockSpec((eb, d), lambda i: (i, 0)),
            pl.BlockSpec((2, d), lambda i: (0, 0)),
            pl.BlockSpec((d, d), lambda i: (0, 0)),
            pl.BlockSpec((1, d), lambda i: (0, 0)),
            pl.BlockSpec((2, d), lambda i: (0, 0)),
        ],
        out_specs=[
            pl.BlockSpec((eb, d), lambda i: (i, 0)),
            pl.BlockSpec((2, d), lambda i: (0, 0)),
        ],
        out_shape=(jax.ShapeDtypeStruct((e, d), jnp.float32),
                   jax.ShapeDtypeStruct((2, d), jnp.float32)),
        scratch_shapes=[pltpu.VMEM((2, d), jnp.float32)],
        interpret=_INTERPRET,
    )(y1, ss1, W2, b2.reshape(1, d), gb)


def _tn_body(h_ref, a0_ref, a1_ref, u1a_ref, u1b_ref, ub1_ref, g1_ref, be1_ref,
             u2_ref, ub2_ref, g2_ref, be2_ref, out_ref):
    n = h_ref.shape[0]
    inv_n = 1.0 / n
    aggr = a0_ref[0:n, :] + a1_ref[0:n, :]
    y = (jnp.dot(h_ref[...], u1a_ref[...], preferred_element_type=jnp.float32)
         + jnp.dot(aggr, u1b_ref[...], preferred_element_type=jnp.float32)
         + ub1_ref[...])
    m = jnp.sum(y, 0, keepdims=True) * inv_n
    v = jnp.sum(y * y, 0, keepdims=True) * inv_n - m * m
    s = g1_ref[...] * lax.rsqrt(v + EPS)
    a = jnp.maximum(y * s + (be1_ref[...] - m * s), 0.0)
    y = (jnp.dot(a, u2_ref[...], preferred_element_type=jnp.float32)
         + ub2_ref[...])
    m = jnp.sum(y, 0, keepdims=True) * inv_n
    v = jnp.sum(y * y, 0, keepdims=True) * inv_n - m * m
    s = g2_ref[...] * lax.rsqrt(v + EPS)
    out_ref[...] = jnp.maximum(y * s + (be2_ref[...] - m * s), 0.0)


def _node_mlp(h, a0, a1, U1a, U1b, ub1, ug1, ube1, U2, ub2, ug2, ube2):
    n, d = h.shape
    r = lambda x: x.reshape(1, d)
    return pl.pallas_call(
        _tn_body,
        out_shape=jax.ShapeDtypeStruct((n, d), jnp.float32),
        interpret=_INTERPRET,
    )(h, a0, a1, U1a, U1b, r(ub1), r(ug1), r(ube1), U2, r(ub2), r(ug2), r(ube2))


_NC, _NS, _L = 2, 16, 16
_NW = _NC * _NS
_C = 80
_NBUF = 5


def _copy_idx_chunk(idx_all, off, dst_row):
    for k in range(_C // _L):
        sl = pl.ds(k * _L, _L)
        dst_row[sl] = idx_all[pl.ds(off + k * _L, _L)]


_NBG = 4

def _scg_body(p_hbm, q_hbm, src_hbm, dst_hbm, t_hbm,
              ia_s, ia_d, ib_s, ib_d, bufp, bufq, sem_g, sem_w):
    e = t_hbm.shape[0]
    d = p_hbm.shape[1]
    per_w = e // _NW
    nchunks = per_w // _C
    wid = lax.axis_index("s") * _NC + lax.axis_index("c")
    w0 = wid * per_w

    pltpu.sync_copy(src_hbm.at[pl.ds(w0, per_w)], ia_s)
    pltpu.sync_copy(dst_hbm.at[pl.ds(w0, per_w)], ia_d)

    def fire_gather(j, b):
        _copy_idx_chunk(ia_s, j * _C, ib_s.at[b])
        _copy_idx_chunk(ia_d, j * _C, ib_d.at[b])
        pltpu.async_copy(p_hbm.at[ib_s.at[b]], bufp.at[b], sem_g.at[b])
        pltpu.async_copy(q_hbm.at[ib_d.at[b]], bufq.at[b], sem_g.at[b])

    def drain_gather(b):
        pltpu.make_async_copy(p_hbm.at[ib_s.at[b]], bufp.at[b],
                              sem_g.at[b]).wait()
        pltpu.make_async_copy(q_hbm.at[ib_d.at[b]], bufq.at[b],
                              sem_g.at[b]).wait()

    def compute(b):
        def row(r, c2):
            for u in range(2):
                for k in range(d // _L):
                    sl = pl.ds(k * _L, _L)
                    bufp[b, 2 * r + u, sl] = (bufp[b, 2 * r + u, sl]
                                              + bufq[b, 2 * r + u, sl])
            return c2

        lax.fori_loop(0, _C // 2, row, 0)

    def fire_write(j, b):
        pltpu.async_copy(bufp.at[b], t_hbm.at[pl.ds(w0 + j * _C, _C)],
                         sem_w.at[b])

    def drain_write(j, b):
        pltpu.make_async_copy(bufp.at[b], t_hbm.at[pl.ds(w0 + j * _C, _C)],
                              sem_w.at[b]).wait()

    for b in range(_NBG - 1):
        fire_gather(b, b)

    def outer(o, carry):
        for b in range(_NBG):
            j = o * _NBG + b
            drain_gather(b)
            compute(b)

            @pl.when(j >= 1)
            def _():
                drain_write(j - 1, (b - 1) % _NBG)

            @pl.when(j + _NBG - 1 < nchunks)
            def _():
                fire_gather(j + _NBG - 1, (b + _NBG - 1) % _NBG)

            fire_write(j, b)
        return carry

    ntail = nchunks % _NBG
    lax.fori_loop(0, nchunks // _NBG, outer, 0)
    for x in range(ntail):
        j = nchunks - ntail + x
        b = j % _NBG
        drain_gather(b)
        compute(b)
        drain_write(j - 1, (b - 1) % _NBG)
        fire_write(j, b)
    drain_write(nchunks - 1, (nchunks - 1) % _NBG)


def _sc_gather_add(p, q, src, dst):
    n, d = p.shape
    e = src.shape[0]
    per_w = e // _NW
    mesh = plsc.VectorSubcoreMesh(core_axis_name="c", subcore_axis_name="s")
    return pl.kernel(
        _scg_body,
        out_type=jax.ShapeDtypeStruct((e, d), jnp.float32),
        mesh=mesh,
        scratch_types=[
            pltpu.VMEM((per_w,), jnp.int32),
            pltpu.VMEM((per_w,), jnp.int32),
            pltpu.VMEM((_NBG, _C), jnp.int32),
            pltpu.VMEM((_NBG, _C), jnp.int32),
            pltpu.VMEM((_NBG, _C, d), jnp.float32),
            pltpu.VMEM((_NBG, _C, d), jnp.float32),
            pltpu.SemaphoreType.DMA((_NBG,)),
            pltpu.SemaphoreType.DMA((_NBG,)),
        ],
    )(p, q, src, dst)


_NBS = 4


def _scs_body(y2_hbm, src_hbm, st_hbm, part_hbm,
              ib_s, buf, stv, aggr_sh, sem_l, sem_sc):
    e = y2_hbm.shape[0]
    np_ = part_hbm.shape[1]
    d = y2_hbm.shape[1]
    per_w = e // _NW
    nchunks = per_w // _C
    rpt = np_ // _NS
    sid = lax.axis_index("s")
    cid = lax.axis_index("c")
    wid = sid * _NC + cid
    w0 = wid * per_w

    zvec = jnp.zeros((_L,), jnp.float32)

    def zrow(r, c2):
        for u in range(2):
            for k in range(d // _L):
                buf[0, 2 * r + u, pl.ds(k * _L, _L)] = zvec
        return c2

    lax.fori_loop(0, _C // 2, zrow, 0)
    for i in range(rpt // _C):
        pltpu.sync_copy(buf.at[0],
                        aggr_sh.at[pl.ds(sid * rpt + i * _C, _C)])
    rem = rpt % _C
    if rem:
        pltpu.sync_copy(buf.at[0, :rem],
                        aggr_sh.at[pl.ds(sid * rpt + rpt - rem, rem)])
    pltpu.sync_copy(st_hbm, stv)
    plsc.subcore_barrier()
    svec = [stv[0, pl.ds(k * _L, _L)] for k in range(d // _L)]
    tvec = [stv[1, pl.ds(k * _L, _L)] for k in range(d // _L)]

    def fire_load(j, b):
        pltpu.async_copy(y2_hbm.at[pl.ds(w0 + j * _C, _C)], buf.at[b],
                         sem_l.at[b])
        pltpu.async_copy(src_hbm.at[pl.ds(w0 + j * _C, _C)], ib_s.at[b],
                         sem_l.at[b])

    def drain_load(j, b):
        pltpu.make_async_copy(y2_hbm.at[pl.ds(w0 + j * _C, _C)],
                              buf.at[b], sem_l.at[b]).wait()
        pltpu.make_async_copy(src_hbm.at[pl.ds(w0 + j * _C, _C)],
                              ib_s.at[b], sem_l.at[b]).wait()

    def compute(b):
        def row(r, c2):
            for u in range(2):
                for k in range(d // _L):
                    sl = pl.ds(k * _L, _L)
                    buf[b, 2 * r + u, sl] = jnp.maximum(
                        buf[b, 2 * r + u, sl] * svec[k] + tvec[k], 0.0)
            return c2

        lax.fori_loop(0, _C // 2, row, 0)

    def drain_scatter(b):
        pltpu.make_async_copy(buf.at[b], aggr_sh.at[ib_s.at[b]],
                              sem_sc).wait()

    for b in range(_NBS - 1):
        fire_load(b, b)

    def outer(o, carry):
        for b in range(_NBS):
            j = o * _NBS + b
            drain_load(j, b)
            compute(b)

            @pl.when(j >= 1)
            def _():
                drain_scatter((b - 1) % _NBS)

            @pl.when(j + _NBS - 1 < nchunks)
            def _():
                fire_load(j + _NBS - 1, (b + _NBS - 1) % _NBS)

            pltpu.async_copy(buf.at[b], aggr_sh.at[ib_s.at[b]], sem_sc,
                             add=True)
        return carry

    ntail = nchunks % _NBS
    lax.fori_loop(0, nchunks // _NBS, outer, 0)
    for t in range(ntail):
        j = nchunks - ntail + t
        b = j % _NBS
        drain_load(j, b)
        compute(b)
        drain_scatter((b - 1) % _NBS)
        pltpu.async_copy(buf.at[b], aggr_sh.at[ib_s.at[b]], sem_sc,
                         add=True)
    drain_scatter((nchunks - 1) % _NBS)
    plsc.subcore_barrier()
    pltpu.sync_copy(aggr_sh.at[pl.ds(sid * rpt, rpt)],
                    part_hbm.at[cid, pl.ds(sid * rpt, rpt)])


def _sc_scatter(y2, src, ss2, n):
    e, d = y2.shape
    np_ = ((n + 8 * _NS - 1) // (8 * _NS)) * (8 * _NS)
    mesh = plsc.VectorSubcoreMesh(core_axis_name="c", subcore_axis_name="s")
    return pl.kernel(
        _scs_body,
        out_type=jax.ShapeDtypeStruct((_NC, np_, d), jnp.float32),
        mesh=mesh,
        scratch_types=[
            pltpu.VMEM((_NBS, _C), jnp.int32),
            pltpu.VMEM((_NBS, _C, d), jnp.float32),
            pltpu.VMEM((2, d), jnp.float32),
            pltpu.VMEM_SHARED((np_, d), jnp.float32),
            pltpu.SemaphoreType.DMA((_NBS,)),
            pltpu.SemaphoreType.DMA,
        ],
    )(y2, src, ss2)


def kernel(h, edge_index, edge_attr, W1, b1, g1, be1, W2, b2, g2, be2,
           U1, ub1, ug1, ube1, U2, ub2, ug2, ube2):
    n, d = h.shape
    e = edge_index.shape[1]
    src = edge_index[0].astype(jnp.int32)
    dst = edge_index[1].astype(jnp.int32)
    W1a, W1b, W1c = W1[:d], W1[d:2 * d], W1[2 * d:]
    EB = 20000

    p, q = _precompute_pq(h, W1a, W1b, b1)
    t = _sc_gather_add(p, q, src, dst)
    y1, ss1 = _pass_a(t, edge_attr, W1c, g1, be1, EB)
    y2, ss2 = _pass_b(y1, ss1, W2, b2, g2, be2, EB)
    part = _sc_scatter(y2, src, ss2, n)
    return _node_mlp(h, part[0], part[1], U1[:d], U1[d:], ub1, ug1, ube1,
                     U2, ub2, ug2, ube2)
